# baseline (device time: 1111694 ns/iter reference)
import jax
import jax.numpy as jnp
from jax import lax
from jax.experimental import pallas as pl
from jax.experimental.pallas import tpu as pltpu

N_Z = 4
NB = 4
S_R1, S_R2, S_L1, S_L2, F_R, F_L = range(6)
NSTREAM = 6


def kernel(x):
    m_per, n = x.shape
    half = m_per // 2
    blk = half // NB

    def body(x_ref, out_ref, local_sem, send_sems, recv_sems):
        my_x = lax.axis_index("x")
        my_y = lax.axis_index("y")
        my_z = lax.axis_index("z")
        y_off = my_y * half
        py_off = (1 - my_y) * half
        last = N_Z - 1

        def at_z(z):
            return (my_x, my_y, z)

        y_partner = (my_x, 1 - my_y, my_z)

        z_streams = {
            S_R1: dict(
                send=my_z <= last - 1, tgt=at_z(jnp.minimum(my_z + 1, last)),
                recv=my_z >= 1, src_z=at_z(jnp.maximum(my_z - 1, 0)),
                origin=jnp.maximum(my_z - 1, 0),
            ),
            S_R2: dict(
                send=my_z <= last - 2, tgt=at_z(jnp.minimum(my_z + 2, last)),
                recv=my_z >= 2, src_z=at_z(jnp.maximum(my_z - 2, 0)),
                origin=jnp.maximum(my_z - 2, 0),
            ),
            S_L1: dict(
                send=my_z >= 1, tgt=at_z(jnp.maximum(my_z - 1, 0)),
                recv=my_z <= last - 1, src_z=at_z(jnp.minimum(my_z + 1, last)),
                origin=jnp.minimum(my_z + 1, last),
            ),
            S_L2: dict(
                send=my_z >= 2, tgt=at_z(jnp.maximum(my_z - 2, 0)),
                recv=my_z <= last - 2, src_z=at_z(jnp.minimum(my_z + 2, last)),
                origin=jnp.minimum(my_z + 2, last),
            ),
            F_R: dict(
                send=my_z == 2, tgt=at_z(last),
                recv=my_z == last, src_z=at_z(2), origin=jnp.int32(0),
            ),
            F_L: dict(
                send=my_z == 1, tgt=at_z(0),
                recv=my_z == 0, src_z=at_z(1), origin=jnp.int32(last),
            ),
        }

        barrier_sem = pltpu.get_barrier_semaphore()
        pl.semaphore_signal(
            barrier_sem, inc=1, device_id=y_partner,
            device_id_type=pl.DeviceIdType.MESH,
        )
        for guard, tgt in (
            (my_z >= 1, at_z(jnp.maximum(my_z - 1, 0))),
            (my_z >= 2, at_z(jnp.maximum(my_z - 2, 0))),
            (my_z <= last - 1, at_z(jnp.minimum(my_z + 1, last))),
            (my_z <= last - 2, at_z(jnp.minimum(my_z + 2, last))),
        ):
            def _sig(tgt=tgt):
                pl.semaphore_signal(
                    barrier_sem, inc=1, device_id=tgt,
                    device_id_type=pl.DeviceIdType.MESH,
                )
            pl.when(guard)(_sig)
        n_wait = (
            1
            + (my_z >= 1).astype(jnp.int32)
            + (my_z >= 2).astype(jnp.int32)
            + (my_z <= last - 1).astype(jnp.int32)
            + (my_z <= last - 2).astype(jnp.int32)
        )
        pl.semaphore_wait(barrier_sem, n_wait)

        local_copy = pltpu.make_async_copy(
            x_ref, out_ref.at[pl.ds(my_z * m_per, m_per)], local_sem
        )
        local_copy.start()

        started = []

        def start(guard, desc):
            pl.when(guard)(desc.start)
            started.append((guard, desc))

        def z_send(sid, b):
            st = z_streams[sid]
            if sid in (S_R1, S_R2, S_L1, S_L2):
                src = x_ref.at[pl.ds(y_off + b * blk, blk)]
                row = my_z * m_per + y_off + b * blk
            else:
                row = st["origin"] * m_per + y_off + b * blk
                src = out_ref.at[pl.ds(row, blk)]
            return st["send"], pltpu.make_async_remote_copy(
                src_ref=src,
                dst_ref=out_ref.at[pl.ds(row, blk)],
                send_sem=send_sems.at[sid, b],
                recv_sem=recv_sems.at[sid, b],
                device_id=st["tgt"],
                device_id_type=pl.DeviceIdType.MESH,
            )

        def z_recv(sid, b):
            st = z_streams[sid]
            row = st["origin"] * m_per + y_off + b * blk
            return st["recv"], pltpu.make_async_remote_copy(
                src_ref=out_ref.at[pl.ds(row, blk)],
                dst_ref=out_ref.at[pl.ds(row, blk)],
                send_sem=send_sems.at[sid, b],
                recv_sem=recv_sems.at[sid, b],
                device_id=st["src_z"],
                device_id_type=pl.DeviceIdType.MESH,
            )

        def y_fwd(sid, b):
            st = z_streams[sid]
            row = st["origin"] * m_per + y_off + b * blk
            return st["recv"], pltpu.make_async_remote_copy(
                src_ref=out_ref.at[pl.ds(row, blk)],
                dst_ref=out_ref.at[pl.ds(row, blk)],
                send_sem=send_sems.at[NSTREAM + sid, b],
                recv_sem=recv_sems.at[NSTREAM + sid, b],
                device_id=y_partner,
                device_id_type=pl.DeviceIdType.MESH,
            )

        def y_recv(sid, b):
            st = z_streams[sid]
            row = st["origin"] * m_per + py_off + b * blk
            return st["recv"], pltpu.make_async_remote_copy(
                src_ref=out_ref.at[pl.ds(row, blk)],
                dst_ref=out_ref.at[pl.ds(row, blk)],
                send_sem=send_sems.at[NSTREAM + sid, b],
                recv_sem=recv_sems.at[NSTREAM + sid, b],
                device_id=y_partner,
                device_id_type=pl.DeviceIdType.MESH,
            )

        for sid in (S_R2, S_L2, S_R1, S_L1):
            for b in range(NB):
                start(*z_send(sid, b))

        for b in range(NB):
            for sid in (S_R2, S_L2, S_R1, S_L1):
                g, d = z_recv(sid, b)
                pl.when(g)(d.wait_recv)
                if sid == S_R2:
                    start(*z_send(F_R, b))
                elif sid == S_L2:
                    start(*z_send(F_L, b))
                start(*y_fwd(sid, b))
        for b in range(NB):
            for sid in (F_R, F_L):
                g, d = z_recv(sid, b)
                pl.when(g)(d.wait_recv)
                start(*y_fwd(sid, b))

        for b in range(NB):
            for sid in range(NSTREAM):
                g, d = y_recv(sid, b)
                pl.when(g)(d.wait_recv)
        for g, d in started:
            pl.when(g)(d.wait_send)
        local_copy.wait()

    return pl.pallas_call(
        body,
        out_shape=jax.ShapeDtypeStruct((N_Z * m_per, n), x.dtype),
        in_specs=[pl.BlockSpec(memory_space=pl.ANY)],
        out_specs=pl.BlockSpec(memory_space=pl.ANY),
        scratch_shapes=[
            pltpu.SemaphoreType.DMA,
            pltpu.SemaphoreType.DMA((2 * NSTREAM, NB)),
            pltpu.SemaphoreType.DMA((2 * NSTREAM, NB)),
        ],
        compiler_params=pltpu.CompilerParams(collective_id=0),
    )(x)


# device time: 861506 ns/iter; 1.2904x vs baseline; 1.2904x over previous
import jax
import jax.numpy as jnp
from jax import lax
from jax.experimental import pallas as pl
from jax.experimental.pallas import tpu as pltpu

N_Z = 4
ENABLE_Y = True
NB = 4
S_R1, S_R2, S_L1, S_L2, F_R, F_L = range(6)
NSTREAM = 6
X_SELF = 2 * NSTREAM


def kernel(x):
    m_per, n = x.shape
    half = m_per // 2
    blk = half // NB

    def body(x_ref, out_ref, local_sem, send_sems, recv_sems):
        my_x = lax.axis_index("x")
        my_y = lax.axis_index("y")
        my_z = lax.axis_index("z")
        y_off = my_y * half
        py_off = (1 - my_y) * half
        last = N_Z - 1

        def at_z(z):
            return (my_x, my_y, z)

        y_partner = (my_x, 1 - my_y, my_z)

        z_streams = {
            S_R1: dict(
                send=my_z <= last - 1, tgt=at_z(jnp.minimum(my_z + 1, last)),
                recv=my_z >= 1, src_z=at_z(jnp.maximum(my_z - 1, 0)),
                origin=jnp.maximum(my_z - 1, 0),
            ),
            S_R2: dict(
                send=my_z <= last - 2, tgt=at_z(jnp.minimum(my_z + 2, last)),
                recv=my_z >= 2, src_z=at_z(jnp.maximum(my_z - 2, 0)),
                origin=jnp.maximum(my_z - 2, 0),
            ),
            S_L1: dict(
                send=my_z >= 1, tgt=at_z(jnp.maximum(my_z - 1, 0)),
                recv=my_z <= last - 1, src_z=at_z(jnp.minimum(my_z + 1, last)),
                origin=jnp.minimum(my_z + 1, last),
            ),
            S_L2: dict(
                send=my_z >= 2, tgt=at_z(jnp.maximum(my_z - 2, 0)),
                recv=my_z <= last - 2, src_z=at_z(jnp.minimum(my_z + 2, last)),
                origin=jnp.minimum(my_z + 2, last),
            ),
            F_R: dict(
                send=my_z == 2, tgt=at_z(last),
                recv=my_z == last, src_z=at_z(2), origin=jnp.int32(0),
            ),
            F_L: dict(
                send=my_z == 1, tgt=at_z(0),
                recv=my_z == 0, src_z=at_z(1), origin=jnp.int32(last),
            ),
        }

        barrier_sem = pltpu.get_barrier_semaphore()
        for always_tgt in (y_partner, (1 - my_x, my_y, my_z)):
            pl.semaphore_signal(
                barrier_sem, inc=1, device_id=always_tgt,
                device_id_type=pl.DeviceIdType.MESH,
            )
        for guard, tgt in (
            (my_z >= 1, at_z(jnp.maximum(my_z - 1, 0))),
            (my_z >= 2, at_z(jnp.maximum(my_z - 2, 0))),
            (my_z <= last - 1, at_z(jnp.minimum(my_z + 1, last))),
            (my_z <= last - 2, at_z(jnp.minimum(my_z + 2, last))),
        ):
            def _sig(tgt=tgt):
                pl.semaphore_signal(
                    barrier_sem, inc=1, device_id=tgt,
                    device_id_type=pl.DeviceIdType.MESH,
                )
            pl.when(guard)(_sig)
        n_wait = (
            2
            + (my_z >= 1).astype(jnp.int32)
            + (my_z >= 2).astype(jnp.int32)
            + (my_z <= last - 1).astype(jnp.int32)
            + (my_z <= last - 2).astype(jnp.int32)
        )
        pl.semaphore_wait(barrier_sem, n_wait)

        x_partner = (1 - my_x, my_y, my_z)
        mblk = m_per // NB
        x_msgs = []
        for b in range(NB):
            sl = pl.ds(my_z * m_per + b * mblk, mblk)
            d = pltpu.make_async_remote_copy(
                src_ref=x_ref.at[pl.ds(b * mblk, mblk)],
                dst_ref=out_ref.at[sl],
                send_sem=send_sems.at[X_SELF, b],
                recv_sem=recv_sems.at[X_SELF, b],
                device_id=x_partner,
                device_id_type=pl.DeviceIdType.MESH,
            )
            d.start()
            x_msgs.append(d)

        started = []

        def start(guard, desc):
            pl.when(guard)(desc.start)
            started.append((guard, desc))

        def z_send(sid, b):
            st = z_streams[sid]
            if sid in (S_R1, S_R2, S_L1, S_L2):
                src = x_ref.at[pl.ds(y_off + b * blk, blk)]
                row = my_z * m_per + y_off + b * blk
            else:
                row = st["origin"] * m_per + y_off + b * blk
                src = out_ref.at[pl.ds(row, blk)]
            return st["send"], pltpu.make_async_remote_copy(
                src_ref=src,
                dst_ref=out_ref.at[pl.ds(row, blk)],
                send_sem=send_sems.at[sid, b],
                recv_sem=recv_sems.at[sid, b],
                device_id=st["tgt"],
                device_id_type=pl.DeviceIdType.MESH,
            )

        def z_recv(sid, b):
            st = z_streams[sid]
            row = st["origin"] * m_per + y_off + b * blk
            return st["recv"], pltpu.make_async_remote_copy(
                src_ref=out_ref.at[pl.ds(row, blk)],
                dst_ref=out_ref.at[pl.ds(row, blk)],
                send_sem=send_sems.at[sid, b],
                recv_sem=recv_sems.at[sid, b],
                device_id=st["src_z"],
                device_id_type=pl.DeviceIdType.MESH,
            )

        def y_fwd(sid, b):
            st = z_streams[sid]
            row = st["origin"] * m_per + y_off + b * blk
            return st["recv"], pltpu.make_async_remote_copy(
                src_ref=out_ref.at[pl.ds(row, blk)],
                dst_ref=out_ref.at[pl.ds(row, blk)],
                send_sem=send_sems.at[NSTREAM + sid, b],
                recv_sem=recv_sems.at[NSTREAM + sid, b],
                device_id=y_partner,
                device_id_type=pl.DeviceIdType.MESH,
            )

        def y_recv(sid, b):
            st = z_streams[sid]
            row = st["origin"] * m_per + py_off + b * blk
            return st["recv"], pltpu.make_async_remote_copy(
                src_ref=out_ref.at[pl.ds(row, blk)],
                dst_ref=out_ref.at[pl.ds(row, blk)],
                send_sem=send_sems.at[NSTREAM + sid, b],
                recv_sem=recv_sems.at[NSTREAM + sid, b],
                device_id=y_partner,
                device_id_type=pl.DeviceIdType.MESH,
            )

        for sid in (S_R2, S_L2, S_R1, S_L1):
            for b in range(NB):
                start(*z_send(sid, b))

        for b in range(NB):
            for sid in (S_R2, S_L2, S_R1, S_L1):
                g, d = z_recv(sid, b)
                pl.when(g)(d.wait_recv)
                if sid == S_R2:
                    start(*z_send(F_R, b))
                elif sid == S_L2:
                    start(*z_send(F_L, b))
                if ENABLE_Y:
                    start(*y_fwd(sid, b))
        for b in range(NB):
            for sid in (F_R, F_L):
                g, d = z_recv(sid, b)
                pl.when(g)(d.wait_recv)
                if ENABLE_Y:
                    start(*y_fwd(sid, b))

        if ENABLE_Y:
            for b in range(NB):
                for sid in range(NSTREAM):
                    g, d = y_recv(sid, b)
                    pl.when(g)(d.wait_recv)
        for d in x_msgs:
            d.wait_recv()
        for g, d in started:
            pl.when(g)(d.wait_send)
        for d in x_msgs:
            d.wait_send()

    return pl.pallas_call(
        body,
        out_shape=jax.ShapeDtypeStruct((N_Z * m_per, n), x.dtype),
        in_specs=[pl.BlockSpec(memory_space=pl.ANY)],
        out_specs=pl.BlockSpec(memory_space=pl.ANY),
        scratch_shapes=[
            pltpu.SemaphoreType.DMA,
            pltpu.SemaphoreType.DMA((2 * NSTREAM + 1, NB)),
            pltpu.SemaphoreType.DMA((2 * NSTREAM + 1, NB)),
        ],
        compiler_params=pltpu.CompilerParams(collective_id=0),
    )(x)
